# baseline (speedup 1.0000x reference)
import functools

import jax
import jax.numpy as jnp
from jax import lax
from jax.experimental import pallas as pl
from jax.experimental.pallas import tpu as pltpu
from jax.experimental.pallas import tpu_sc as plsc

_ROWS = 64
_COLS = 1000000
_CHUNK = 3584
_NCHUNK = 164
_SC_COLS = _NCHUNK * _CHUNK
_VECS = _CHUNK // 16
_PER_SHARD = 41
_NUM_CORES = 2
_NUM_SUBCORES = 16
_INT_MAX = 2**31 - 1
_TC_G = (_COLS - _SC_COLS + _CHUNK - 1) // _CHUNK

_mesh = plsc.VectorSubcoreMesh(
    core_axis_name="c", subcore_axis_name="s",
    num_cores=_NUM_CORES, num_subcores=_NUM_SUBCORES,
)


def _scan_chunk(buf, vec_base, carry):
    def body(i, c):
        ib = lax.broadcast_in_dim(vec_base + i, (16,), ())
        out = []
        for r in range(8):
            bv, bn = c[2 * r], c[2 * r + 1]
            v = buf[r, pl.ds(i * 16, 16)]
            m = v > bv
            out.append(jnp.where(m, v, bv))
            out.append(jnp.where(m, ib, bn))
        return tuple(out)

    return plsc.parallel_loop(0, _VECS, step=1, unroll=2, carry=carry)(body)


_KERNEL_KWARGS = dict(
    out_type=(jax.ShapeDtypeStruct((32, 16), jnp.float32),
              jax.ShapeDtypeStruct((32, 16), jnp.int32)),
    mesh=_mesh,
    scratch_types=[
        pltpu.VMEM((8, _CHUNK), jnp.float32),
        pltpu.VMEM((8, _CHUNK), jnp.float32),
        pltpu.VMEM((8, _CHUNK), jnp.float32),
        pltpu.VMEM((8, _CHUNK), jnp.float32),
        pltpu.VMEM((16,), jnp.float32),
        pltpu.VMEM((16,), jnp.int32),
        pltpu.SemaphoreType.DMA,
        pltpu.SemaphoreType.DMA,
        pltpu.SemaphoreType.DMA,
        pltpu.SemaphoreType.DMA,
    ],
    compiler_params=pltpu.CompilerParams(needs_layout_passes=False),
)


def _argmax_body(x_hbm, oval_hbm, oidx_hbm, buf0, buf1, buf2, buf3,
                 val_v, idx_v, sem0, sem1, sem2, sem3):
    c = lax.axis_index("c")
    s = lax.axis_index("s")
    wid = c * 16 + s
    g = c * 4 + s // 4
    sh = s % 4
    row0 = pl.multiple_of(g * 8, 8)
    lane = lax.iota(jnp.int32, 16)

    def chunk_src(k):
        cn = jnp.minimum(sh + 4 * k, _NCHUNK - 1)
        col = pl.multiple_of(cn * _CHUNK, _CHUNK)
        return x_hbm.at[pl.ds(row0, 8), pl.ds(col, _CHUNK)], cn

    bufs = (buf0, buf1, buf2, buf3)
    sems = (sem0, sem1, sem2, sem3)

    for b in range(4):
        src, _ = chunk_src(b)
        pltpu.async_copy(src, bufs[b], sems[b])

    neg_inf = jnp.full((16,), -jnp.inf, jnp.float32)
    zero = jnp.zeros((16,), jnp.int32)
    carry = (neg_inf, zero) * 8

    def quad(p, carry):
        for b in range(4):
            k = 4 * p + b
            src, cn = chunk_src(k)
            pltpu.make_async_copy(src, bufs[b], sems[b]).wait()
            carry = _scan_chunk(bufs[b], cn * _VECS, carry)

            @pl.when(k + 4 < _PER_SHARD)
            def _(k=k, b=b):
                src, _ = chunk_src(k + 4)
                pltpu.async_copy(src, bufs[b], sems[b])

        return carry

    carry = lax.fori_loop(0, _PER_SHARD // 4, quad, carry)

    for b in range(_PER_SHARD % 4):
        k = (_PER_SHARD // 4) * 4 + b
        src, cn = chunk_src(k)
        pltpu.make_async_copy(src, bufs[b], sems[b]).wait()
        carry = _scan_chunk(bufs[b], cn * _VECS, carry)
    carry = list(carry)

    res_val = jnp.full((16,), -jnp.inf, jnp.float32)
    res_idx = jnp.zeros((16,), jnp.int32)
    for r in range(8):
        bv, bn = carry[2 * r], carry[2 * r + 1]
        idx = (bn << 4) + lane
        mx = jnp.max(bv)
        cand = jnp.where(bv == mx, idx, jnp.int32(_INT_MAX))
        ii = jnp.min(cand)
        res_val = jnp.where(lane == r, mx, res_val)
        res_idx = jnp.where(lane == r, ii, res_idx)

    val_v[...] = res_val
    idx_v[...] = res_idx
    pltpu.sync_copy(val_v, oval_hbm.at[wid])
    pltpu.sync_copy(idx_v, oidx_hbm.at[wid])


_argmax_sc = pl.kernel(_argmax_body, **_KERNEL_KWARGS)


def _tc_scan_body(x_ref, val_ref, idx_ref, rv, ri):
    i = pl.program_id(0)

    @pl.when(i == 0)
    def _():
        rv[...] = jnp.full((64, 1), -jnp.inf, jnp.float32)
        ri[...] = jnp.zeros((64, 1), jnp.int32)

    base = _SC_COLS + i * _CHUNK
    cols = lax.broadcasted_iota(jnp.int32, (64, _CHUNK), 1) + base
    v = x_ref[...]
    v = jnp.where(cols < _COLS, v, -jnp.inf)
    bm = jnp.max(v, axis=1, keepdims=True)
    cand = jnp.where(v == bm, cols, jnp.int32(_INT_MAX))
    bi = jnp.min(cand, axis=1, keepdims=True)
    m = bm > rv[...]
    rv[...] = jnp.where(m, bm, rv[...])
    ri[...] = jnp.where(m, bi, ri[...])

    @pl.when(i == _TC_G - 1)
    def _():
        val_ref[...] = rv[...]
        idx_ref[...] = ri[...]


_tc_scan = pl.pallas_call(
    _tc_scan_body,
    grid=(_TC_G,),
    in_specs=[pl.BlockSpec((64, _CHUNK), lambda i: (0, _NCHUNK + i))],
    out_specs=(pl.BlockSpec((64, 1), lambda i: (0, 0)),
               pl.BlockSpec((64, 1), lambda i: (0, 0))),
    out_shape=(jax.ShapeDtypeStruct((64, 1), jnp.float32),
               jax.ShapeDtypeStruct((64, 1), jnp.int32)),
    scratch_shapes=[pltpu.VMEM((64, 1), jnp.float32),
                    pltpu.VMEM((64, 1), jnp.int32)],
)


def _merge_body(val_ref, idx_ref, tval_ref, tidx_ref, out_ref):
    for g in range(8):
        base = (g // 4) * 16 + (g % 4) * 4
        bv = val_ref[base]
        bi = idx_ref[base]
        parts = [(val_ref[base + k], idx_ref[base + k]) for k in range(1, 4)]
        parts.append((tval_ref[g], tidx_ref[g]))
        for ov, oi in parts:
            take = (ov > bv) | ((ov == bv) & (oi < bi))
            bv = jnp.where(take, ov, bv)
            bi = jnp.where(take, oi, bi)
        out_ref[g] = bi


_merge_tc = pl.pallas_call(
    _merge_body,
    out_shape=jax.ShapeDtypeStruct((8, 16), jnp.int32),
)


def kernel(inputs):
    pval, pidx = _argmax_sc(inputs)
    tval, tidx = _tc_scan(inputs)
    tv2 = jnp.concatenate(
        [tval.reshape(8, 8), jnp.full((8, 8), -jnp.inf, jnp.float32)], axis=1)
    ti2 = jnp.concatenate(
        [tidx.reshape(8, 8), jnp.full((8, 8), _INT_MAX, jnp.int32)], axis=1)
    merged = _merge_tc(pval, pidx, tv2, ti2)
    return merged[:, :8].reshape(_ROWS).astype(jnp.int64)


# speedup vs baseline: 1.2274x; 1.2274x over previous
import jax
import jax.numpy as jnp
from jax import lax
from jax.experimental import pallas as pl
from jax.experimental.pallas import tpu as pltpu
from jax.experimental.pallas import tpu_sc as plsc

_ROWS = 64
_COLS = 1000000
_CHUNK = 3584
_MAIN = 999936
_NCHUNK = _MAIN // _CHUNK
_TAIL = _COLS - _MAIN
_VECS = _CHUNK // 16
_PER_SHARD = 70
_NUM_CORES = 2
_NUM_SUBCORES = 16
_INT_MAX = 2**31 - 1

_mesh = plsc.VectorSubcoreMesh(
    core_axis_name="c", subcore_axis_name="s",
    num_cores=_NUM_CORES, num_subcores=_NUM_SUBCORES,
)


def _scan_chunk(buf, vec_base, carry):
    def body(i, c):
        ib = lax.broadcast_in_dim(vec_base + i, (16,), ())
        out = []
        for r in range(8):
            bv, bn = c[2 * r], c[2 * r + 1]
            v = buf[r, pl.ds(i * 16, 16)]
            m = v > bv
            out.append(jnp.where(m, v, bv))
            out.append(jnp.where(m, ib, bn))
        return tuple(out)

    return plsc.parallel_loop(0, _VECS, step=1, unroll=2, carry=carry)(body)


_KERNEL_KWARGS = dict(
    out_type=(jax.ShapeDtypeStruct((32, 16), jnp.float32),
              jax.ShapeDtypeStruct((32, 16), jnp.int32)),
    mesh=_mesh,
    scratch_types=[
        pltpu.VMEM((8, _CHUNK), jnp.float32),
        pltpu.VMEM((8, _CHUNK), jnp.float32),
        pltpu.VMEM((8, _CHUNK), jnp.float32),
        pltpu.VMEM((8, _CHUNK), jnp.float32),
        pltpu.VMEM((8, _TAIL), jnp.float32),
        pltpu.VMEM((16,), jnp.float32),
        pltpu.VMEM((16,), jnp.int32),
        pltpu.SemaphoreType.DMA,
        pltpu.SemaphoreType.DMA,
        pltpu.SemaphoreType.DMA,
        pltpu.SemaphoreType.DMA,
        pltpu.SemaphoreType.DMA,
    ],
    compiler_params=pltpu.CompilerParams(needs_layout_passes=False),
)


def _argmax_body(x_hbm, oval_hbm, oidx_hbm, buf0, buf1, buf2, buf3, tailbuf,
                 val_v, idx_v, sem0, sem1, sem2, sem3, semt):
    c = lax.axis_index("c")
    s = lax.axis_index("s")
    wid = c * 16 + s
    g = c * 4 + s // 4
    sh = s % 4
    row0 = pl.multiple_of(g * 8, 8)
    lane = lax.iota(jnp.int32, 16)

    def chunk_src(k):
        cn = jnp.minimum(sh + 4 * k, _NCHUNK - 1)
        col = pl.multiple_of(cn * _CHUNK, _CHUNK)
        return x_hbm.at[pl.ds(row0, 8), pl.ds(col, _CHUNK)], cn

    bufs = (buf0, buf1, buf2, buf3)
    sems = (sem0, sem1, sem2, sem3)

    pltpu.async_copy(
        x_hbm.at[pl.ds(row0, 8), pl.ds(_MAIN, _TAIL)], tailbuf, semt)
    for b in range(4):
        src, _ = chunk_src(b)
        pltpu.async_copy(src, bufs[b], sems[b])

    neg_inf = jnp.full((16,), -jnp.inf, jnp.float32)
    zero = jnp.zeros((16,), jnp.int32)
    carry = (neg_inf, zero) * 8

    def quad(p, carry):
        for b in range(4):
            k = 4 * p + b
            src, cn = chunk_src(k)
            pltpu.make_async_copy(src, bufs[b], sems[b]).wait()
            carry = _scan_chunk(bufs[b], cn * _VECS, carry)

            @pl.when(k + 4 < _PER_SHARD)
            def _(k=k, b=b):
                src, _ = chunk_src(k + 4)
                pltpu.async_copy(src, bufs[b], sems[b])

        return carry

    carry = lax.fori_loop(0, _PER_SHARD // 4, quad, carry)

    for b in range(_PER_SHARD % 4):
        k = (_PER_SHARD // 4) * 4 + b
        src, cn = chunk_src(k)
        pltpu.make_async_copy(src, bufs[b], sems[b]).wait()
        carry = _scan_chunk(bufs[b], cn * _VECS, carry)

    pltpu.make_async_copy(
        x_hbm.at[pl.ds(row0, 8), pl.ds(_MAIN, _TAIL)], tailbuf, semt).wait()
    carry = list(carry)
    for r in range(8):
        bv, bn = carry[2 * r], carry[2 * r + 1]
        for i in range(_TAIL // 16):
            v = tailbuf[r, pl.ds(i * 16, 16)]
            ib = jnp.full((16,), _MAIN // 16 + i, jnp.int32)
            m = v > bv
            bv = jnp.where(m, v, bv)
            bn = jnp.where(m, ib, bn)
        carry[2 * r], carry[2 * r + 1] = bv, bn

    res_val = jnp.full((16,), -jnp.inf, jnp.float32)
    res_idx = jnp.zeros((16,), jnp.int32)
    for r in range(8):
        bv, bn = carry[2 * r], carry[2 * r + 1]
        idx = (bn << 4) + lane
        mx = jnp.max(bv)
        cand = jnp.where(bv == mx, idx, jnp.int32(_INT_MAX))
        ii = jnp.min(cand)
        res_val = jnp.where(lane == r, mx, res_val)
        res_idx = jnp.where(lane == r, ii, res_idx)

    val_v[...] = res_val
    idx_v[...] = res_idx
    pltpu.sync_copy(val_v, oval_hbm.at[wid])
    pltpu.sync_copy(idx_v, oidx_hbm.at[wid])


_argmax_sc = pl.kernel(_argmax_body, **_KERNEL_KWARGS)


def _merge_body(val_ref, idx_ref, out_ref):
    for g in range(8):
        base = (g // 4) * 16 + (g % 4) * 4
        bv = val_ref[base]
        bi = idx_ref[base]
        for k in range(1, 4):
            ov = val_ref[base + k]
            oi = idx_ref[base + k]
            take = (ov > bv) | ((ov == bv) & (oi < bi))
            bv = jnp.where(take, ov, bv)
            bi = jnp.where(take, oi, bi)
        out_ref[g] = bi


_merge_tc = pl.pallas_call(
    _merge_body,
    out_shape=jax.ShapeDtypeStruct((8, 16), jnp.int32),
)


def kernel(inputs):
    pval, pidx = _argmax_sc(inputs)
    merged = _merge_tc(pval, pidx)
    return merged[:, :8].reshape(_ROWS).astype(jnp.int64)
